# baseline (speedup 1.0000x reference)
import functools
import jax
import jax.numpy as jnp
from jax import lax
from jax.experimental import pallas as pl
from jax.experimental.pallas import tpu as pltpu
from jax.experimental.pallas import tpu_sc as plsc

_B = 16384
_N = 1003
_NOUT = 1000
_NPAD = 1008
_NW = 32
_RPW = _B // _NW
_CHUNK = 16
_NCH = _RPW // _CHUNK

_LN2 = 0.6931471805599453
_SQRT2 = 1.4142135623730951


def _vlog(v):
    bits = lax.bitcast_convert_type(v, jnp.int32)
    e = (bits >> 23) - 127
    m = lax.bitcast_convert_type((bits & 0x007FFFFF) | 0x3F800000, jnp.float32)
    big = m > _SQRT2
    m = jnp.where(big, m * 0.5, m)
    e = e + jnp.where(big, 1, 0)
    t = (m - 1.0) / (m + 1.0)
    t2 = t * t
    p = t * (2.0 + t2 * (2.0 / 3.0 + t2 * (2.0 / 5.0 + t2 * (2.0 / 7.0 + t2 * (2.0 / 9.0)))))
    return e.astype(jnp.float32) * _LN2 + p


def _sc_kernel_body(x_hbm, t_hbm, b_hbm, out_hbm, xbuf, obuf, tbuf, bbuf):
    wid = lax.axis_index("s") * 2 + lax.axis_index("c")
    pltpu.sync_copy(t_hbm, tbuf)
    pltpu.sync_copy(b_hbm, bbuf)
    lane = lax.iota(jnp.int32, 16)

    def row_body(r, _):
        acc1 = jnp.zeros((16,), jnp.float32)
        acc2 = jnp.zeros((16,), jnp.float32)
        acc3 = jnp.zeros((16,), jnp.float32)
        el = jnp.zeros((16,), jnp.float32)

        base = r * _N

        def zchunk(off):
            xv = xbuf[pl.ds(base + off, 16)]
            tv = tbuf[pl.ds(off, 16)]
            bv = bbuf[pl.ds(off, 16)]
            return xv * tv + bv

        for k in range(62):
            off = 16 * k
            e = jnp.exp(zchunk(off))
            if k < 24:
                acc1 = acc1 + e
            elif k == 24:
                acc1 = acc1 + jnp.where(lane < 8, e, 0.0)
                acc2 = acc2 + jnp.where(lane >= 8, e, 0.0)
                el = el + jnp.where(lane == 7, e, 0.0)
            elif k < 54:
                acc2 = acc2 + e
            elif k == 54:
                acc2 = acc2 + jnp.where(lane < 2, e, 0.0)
                acc3 = acc3 + jnp.where(lane >= 2, e, 0.0)
                el = el + jnp.where(lane == 1, e, 0.0)
            else:
                acc3 = acc3 + e
        e = jnp.exp(zchunk(987))
        acc3 = acc3 + jnp.where(lane >= 5, e, 0.0)
        el = el + jnp.where(lane == 15, e, 0.0)

        s1 = jnp.sum(acc1)
        s2 = jnp.sum(acc2)
        s3 = jnp.sum(acc3)
        sv = jnp.where(lane == 7, s1, jnp.where(lane == 1, s2, s3))
        renorm = 3.0 - jnp.sum(el / sv)
        packed = jnp.where(lane == 0, s1 * renorm,
                           jnp.where(lane == 1, s2 * renorm, s3 * renorm))
        cvec = _vlog(packed)
        c1 = jnp.sum(jnp.where(lane == 0, cvec, 0.0))
        c2 = jnp.sum(jnp.where(lane == 1, cvec, 0.0))
        c3 = jnp.sum(jnp.where(lane == 2, cvec, 0.0))

        obase = r * _NOUT
        for k in range(62):
            oof = 16 * k
            if k < 24:
                obuf[pl.ds(obase + oof, 16)] = zchunk(oof) - c1
            elif k == 24:
                za = zchunk(384) - c1
                zb = zchunk(385) - c2
                obuf[pl.ds(obase + 384, 16)] = jnp.where(lane < 7, za, zb)
            elif k < 54:
                obuf[pl.ds(obase + oof, 16)] = zchunk(oof + 1) - c2
            else:
                obuf[pl.ds(obase + oof, 16)] = zchunk(oof + 2) - c3
        obuf[pl.ds(obase + 984, 16)] = zchunk(986) - c3
        return _

    def chunk_body(g, _):
        row0 = wid * _RPW + g * _CHUNK
        pltpu.sync_copy(x_hbm.at[pl.ds(row0 * _N, _CHUNK * _N)], xbuf)
        lax.fori_loop(0, _CHUNK, row_body, 0)
        pltpu.sync_copy(obuf, out_hbm.at[pl.ds(row0 * _NOUT, _CHUNK * _NOUT)])
        return _

    lax.fori_loop(0, _NCH, chunk_body, 0)


@jax.jit
def _run_sc(x, t, b):
    mesh = plsc.VectorSubcoreMesh(core_axis_name="c", subcore_axis_name="s")
    f = functools.partial(
        pl.kernel,
        mesh=mesh,
        compiler_params=pltpu.CompilerParams(needs_layout_passes=False),
        out_type=jax.ShapeDtypeStruct((_B * _NOUT,), jnp.float32),
        scratch_types=[
            pltpu.VMEM((_CHUNK * _N,), jnp.float32),
            pltpu.VMEM((_CHUNK * _NOUT,), jnp.float32),
            pltpu.VMEM((_NPAD,), jnp.float32),
            pltpu.VMEM((_NPAD,), jnp.float32),
        ],
    )(_sc_kernel_body)
    return f(x.reshape(-1), t, b).reshape(_B, _NOUT)


def kernel(x, manyshotTemp, mediumshotTemp, fewshotTemp, manyshotBias,
           mediumshotBias, fewshotBias, many_mask, med_mask, few_mask):
    pad = jnp.zeros((1, _NPAD - _N), jnp.float32)
    t = jnp.concatenate([manyshotTemp, mediumshotTemp, fewshotTemp, pad], axis=1)[0]
    b = jnp.concatenate([manyshotBias, mediumshotBias, fewshotBias, pad], axis=1)[0]
    return _run_sc(x, t, b)


# speedup vs baseline: 1.1787x; 1.1787x over previous
import functools
import jax
import jax.numpy as jnp
from jax import lax
from jax.experimental import pallas as pl
from jax.experimental.pallas import tpu as pltpu
from jax.experimental.pallas import tpu_sc as plsc

_B = 16384
_N = 1003
_NOUT = 1000
_NPAD = 1008
_NW = 32
_RPW = _B // _NW
_CHUNK = 16
_NCH = _RPW // _CHUNK

_LN2 = 0.6931471805599453
_SQRT2 = 1.4142135623730951


def _vlog(v):
    bits = lax.bitcast_convert_type(v, jnp.int32)
    e = (bits >> 23) - 127
    m = lax.bitcast_convert_type((bits & 0x007FFFFF) | 0x3F800000, jnp.float32)
    big = m > _SQRT2
    m = jnp.where(big, m * 0.5, m)
    e = e + jnp.where(big, 1, 0)
    t = (m - 1.0) / (m + 1.0)
    t2 = t * t
    p = t * (2.0 + t2 * (2.0 / 3.0 + t2 * (2.0 / 5.0 + t2 * (2.0 / 7.0 + t2 * (2.0 / 9.0)))))
    return e.astype(jnp.float32) * _LN2 + p


def _sc_kernel_body(x_hbm, t_hbm, b_hbm, out_hbm,
                    xb0, xb1, ob0, ob1, tbuf, bbuf,
                    isem0, isem1, osem0, osem1):
    wid = lax.axis_index("s") * 2 + lax.axis_index("c")
    pltpu.sync_copy(t_hbm, tbuf)
    pltpu.sync_copy(b_hbm, bbuf)
    lane = lax.iota(jnp.int32, 16)
    xbufs = (xb0, xb1)
    obufs = (ob0, ob1)
    isems = (isem0, isem1)
    osems = (osem0, osem1)

    def in_src(g):
        row0 = (wid * _RPW + g * _CHUNK) * _N
        return x_hbm.at[pl.ds(row0, _CHUNK * _N)]

    def out_dst(g):
        row0 = (wid * _RPW + g * _CHUNK) * _NOUT
        return out_hbm.at[pl.ds(row0, _CHUNK * _NOUT)]

    def compute_chunk(xbuf, obuf):
        @plsc.parallel_loop(0, _CHUNK, 1, unroll=2)
        def row_body(r):
            acc1a = jnp.zeros((16,), jnp.float32)
            acc1b = jnp.zeros((16,), jnp.float32)
            acc2a = jnp.zeros((16,), jnp.float32)
            acc2b = jnp.zeros((16,), jnp.float32)
            acc3a = jnp.zeros((16,), jnp.float32)
            acc3b = jnp.zeros((16,), jnp.float32)
            el = jnp.zeros((16,), jnp.float32)
            base = r * _N
            obase = r * _NOUT

            def zchunk(off):
                xv = xbuf[pl.ds(base + off, 16)]
                tv = tbuf[pl.ds(off, 16)]
                bv = bbuf[pl.ds(off, 16)]
                return xv * tv + bv

            for k in range(62):
                z = zchunk(16 * k)
                e = jnp.exp(z)
                if k < 24:
                    obuf[pl.ds(obase + 16 * k, 16)] = z
                    if k % 2 == 0:
                        acc1a = acc1a + e
                    else:
                        acc1b = acc1b + e
                elif k == 24:
                    plsc.store_scatter(obuf, [obase + 384 + lane], z,
                                       mask=lane < 7)
                    plsc.store_scatter(obuf, [obase + 383 + lane], z,
                                       mask=lane >= 8)
                    acc1a = acc1a + jnp.where(lane < 8, e, 0.0)
                    acc2a = acc2a + jnp.where(lane >= 8, e, 0.0)
                    el = el + jnp.where(lane == 7, e, 0.0)
                elif k < 54:
                    obuf[pl.ds(obase + 16 * k - 1, 16)] = z
                    if k % 2 == 0:
                        acc2a = acc2a + e
                    else:
                        acc2b = acc2b + e
                elif k == 54:
                    idx = jnp.where(lane == 0, obase + 863, obase + 862 + lane)
                    plsc.store_scatter(obuf, [idx], z, mask=lane != 1)
                    acc2a = acc2a + jnp.where(lane < 2, e, 0.0)
                    acc3a = acc3a + jnp.where(lane >= 2, e, 0.0)
                    el = el + jnp.where(lane == 1, e, 0.0)
                else:
                    obuf[pl.ds(obase + 16 * k - 2, 16)] = z
                    if k % 2 == 0:
                        acc3a = acc3a + e
                    else:
                        acc3b = acc3b + e
            z = zchunk(987)
            e = jnp.exp(z)
            plsc.store_scatter(obuf, [obase + 985 + lane], z,
                               mask=(lane >= 5) & (lane <= 14))
            acc3a = acc3a + jnp.where(lane >= 5, e, 0.0)
            el = el + jnp.where(lane == 15, e, 0.0)

            s1 = jnp.sum(acc1a + acc1b)
            s2 = jnp.sum(acc2a + acc2b)
            s3 = jnp.sum(acc3a + acc3b)
            sv = jnp.where(lane == 7, s1, jnp.where(lane == 1, s2, s3))
            renorm = 3.0 - jnp.sum(el / sv)
            packed = jnp.where(lane == 0, s1 * renorm,
                               jnp.where(lane == 1, s2 * renorm, s3 * renorm))
            cvec = _vlog(packed)
            c1 = jnp.sum(jnp.where(lane == 0, cvec, 0.0))
            c2 = jnp.sum(jnp.where(lane == 1, cvec, 0.0))
            c3 = jnp.sum(jnp.where(lane == 2, cvec, 0.0))

            for k in range(62):
                p = obase + 16 * k
                v = obuf[pl.ds(p, 16)]
                if k < 24:
                    obuf[pl.ds(p, 16)] = v - c1
                elif k == 24:
                    obuf[pl.ds(p, 16)] = v - jnp.where(lane < 7, c1, c2)
                elif k < 54:
                    obuf[pl.ds(p, 16)] = v - c2
                else:
                    obuf[pl.ds(p, 16)] = v - c3
            p = obase + 984
            v = obuf[pl.ds(p, 16)]
            plsc.store_scatter(obuf, [p + lane], v - c3, mask=lane >= 8)

    pltpu.async_copy(in_src(0), xb0, isem0)
    pltpu.async_copy(in_src(1), xb1, isem1)

    def outer(gg, carry):
        for par in range(2):
            g = gg * 2 + par
            xbuf, obuf = xbufs[par], obufs[par]
            isem, osem = isems[par], osems[par]
            pltpu.make_async_copy(in_src(g), xbuf, isem).wait()

            @pl.when(gg >= 1)
            def _():
                pltpu.make_async_copy(obuf, out_dst(g), osem).wait()

            compute_chunk(xbuf, obuf)
            pltpu.async_copy(obuf, out_dst(g), osem)

            @pl.when(gg < (_NCH // 2 - 1))
            def _():
                pltpu.async_copy(in_src(g + 2), xbuf, isem)
        return carry

    lax.fori_loop(0, _NCH // 2, outer, 0)
    pltpu.make_async_copy(ob0, out_dst(_NCH - 2), osem0).wait()
    pltpu.make_async_copy(ob1, out_dst(_NCH - 1), osem1).wait()


@jax.jit
def _run_sc(x, t, b):
    mesh = plsc.VectorSubcoreMesh(core_axis_name="c", subcore_axis_name="s")
    f = functools.partial(
        pl.kernel,
        mesh=mesh,
        compiler_params=pltpu.CompilerParams(needs_layout_passes=False),
        out_type=jax.ShapeDtypeStruct((_B * _NOUT,), jnp.float32),
        scratch_types=[
            pltpu.VMEM((_CHUNK * _N,), jnp.float32),
            pltpu.VMEM((_CHUNK * _N,), jnp.float32),
            pltpu.VMEM((_CHUNK * _NOUT,), jnp.float32),
            pltpu.VMEM((_CHUNK * _NOUT,), jnp.float32),
            pltpu.VMEM((_NPAD,), jnp.float32),
            pltpu.VMEM((_NPAD,), jnp.float32),
            pltpu.SemaphoreType.DMA,
            pltpu.SemaphoreType.DMA,
            pltpu.SemaphoreType.DMA,
            pltpu.SemaphoreType.DMA,
        ],
    )(_sc_kernel_body)
    return f(x.reshape(-1), t, b).reshape(_B, _NOUT)


def kernel(x, manyshotTemp, mediumshotTemp, fewshotTemp, manyshotBias,
           mediumshotBias, fewshotBias, many_mask, med_mask, few_mask):
    pad = jnp.zeros((1, _NPAD - _N), jnp.float32)
    t = jnp.concatenate([manyshotTemp, mediumshotTemp, fewshotTemp, pad], axis=1)[0]
    b = jnp.concatenate([manyshotBias, mediumshotBias, fewshotBias, pad], axis=1)[0]
    return _run_sc(x, t, b)


# speedup vs baseline: 1.5370x; 1.3039x over previous
import functools
import jax
import jax.numpy as jnp
from jax import lax
from jax.experimental import pallas as pl
from jax.experimental.pallas import tpu as pltpu
from jax.experimental.pallas import tpu_sc as plsc

_B = 16384
_N = 1003
_NOUT = 1000
_NPAD = 1008
_NW = 32
_RPW = _B // _NW
_CHUNK = 16
_NCH = _RPW // _CHUNK

_LN2 = 0.6931471805599453
_SQRT2 = 1.4142135623730951


def _vlog(v):
    bits = lax.bitcast_convert_type(v, jnp.int32)
    e = (bits >> 23) - 127
    m = lax.bitcast_convert_type((bits & 0x007FFFFF) | 0x3F800000, jnp.float32)
    big = m > _SQRT2
    m = jnp.where(big, m * 0.5, m)
    e = e + jnp.where(big, 1, 0)
    t = (m - 1.0) / (m + 1.0)
    t2 = t * t
    p = t * (2.0 + t2 * (2.0 / 3.0 + t2 * (2.0 / 5.0 + t2 * (2.0 / 7.0 + t2 * (2.0 / 9.0)))))
    return e.astype(jnp.float32) * _LN2 + p


_A_OFFS = [16 * k for k in range(62)] + [987]


def _sc_kernel_body(x_hbm, t_hbm, b_hbm, out_hbm,
                    xb0, xb1, ob0, ob1, tbuf, bbuf,
                    isem0, isem1, osem0, osem1):
    wid = lax.axis_index("s") * 2 + lax.axis_index("c")
    pltpu.sync_copy(t_hbm, tbuf)
    pltpu.sync_copy(b_hbm, bbuf)
    lane = lax.iota(jnp.int32, 16)
    xbufs = (xb0, xb1)
    obufs = (ob0, ob1)
    isems = (isem0, isem1)
    osems = (osem0, osem1)

    def in_src(g):
        row0 = (wid * _RPW + g * _CHUNK) * _N
        return x_hbm.at[pl.ds(row0, _CHUNK * _N)]

    def out_dst(g):
        row0 = (wid * _RPW + g * _CHUNK) * _NOUT
        return out_hbm.at[pl.ds(row0, _CHUNK * _NOUT)]

    def compute_chunk(xbuf, obuf):
        @plsc.parallel_loop(0, _CHUNK // 2, 1, unroll=1)
        def pair_body(q):
            baseA = q * _N
            baseB = (q + 8) * _N
            obaseA = q * _NOUT
            obaseB = (q + 8) * _NOUT

            def lds(k):
                off = _A_OFFS[k]
                return (tbuf[pl.ds(off, 16)], bbuf[pl.ds(off, 16)],
                        xbuf[pl.ds(baseA + off, 16)],
                        xbuf[pl.ds(baseB + off, 16)])

            accA = [jnp.zeros((16,), jnp.float32) for _ in range(3)]
            accB = [jnp.zeros((16,), jnp.float32) for _ in range(3)]
            elA = jnp.zeros((16,), jnp.float32)
            elB = jnp.zeros((16,), jnp.float32)

            cur = lds(0)
            for k in range(63):
                nxt = lds(k + 1) if k < 62 else cur
                tv, bv, xa, xc = cur
                za = xa * tv + bv
                zb = xc * tv + bv
                ea = jnp.exp(za)
                eb = jnp.exp(zb)
                if k < 24:
                    obuf[pl.ds(obaseA + 16 * k, 16)] = za
                    obuf[pl.ds(obaseB + 16 * k, 16)] = zb
                    accA[0] = accA[0] + ea
                    accB[0] = accB[0] + eb
                elif k == 24:
                    plsc.store_scatter(obuf, [obaseA + 384 + lane], za,
                                       mask=lane < 7)
                    plsc.store_scatter(obuf, [obaseA + 383 + lane], za,
                                       mask=lane >= 8)
                    plsc.store_scatter(obuf, [obaseB + 384 + lane], zb,
                                       mask=lane < 7)
                    plsc.store_scatter(obuf, [obaseB + 383 + lane], zb,
                                       mask=lane >= 8)
                    accA[0] = accA[0] + jnp.where(lane < 8, ea, 0.0)
                    accA[1] = accA[1] + jnp.where(lane >= 8, ea, 0.0)
                    accB[0] = accB[0] + jnp.where(lane < 8, eb, 0.0)
                    accB[1] = accB[1] + jnp.where(lane >= 8, eb, 0.0)
                    elA = elA + jnp.where(lane == 7, ea, 0.0)
                    elB = elB + jnp.where(lane == 7, eb, 0.0)
                elif k < 54:
                    obuf[pl.ds(obaseA + 16 * k - 1, 16)] = za
                    obuf[pl.ds(obaseB + 16 * k - 1, 16)] = zb
                    accA[1] = accA[1] + ea
                    accB[1] = accB[1] + eb
                elif k == 54:
                    idxA = jnp.where(lane == 0, obaseA + 863, obaseA + 862 + lane)
                    idxB = jnp.where(lane == 0, obaseB + 863, obaseB + 862 + lane)
                    plsc.store_scatter(obuf, [idxA], za, mask=lane != 1)
                    plsc.store_scatter(obuf, [idxB], zb, mask=lane != 1)
                    accA[1] = accA[1] + jnp.where(lane < 2, ea, 0.0)
                    accA[2] = accA[2] + jnp.where(lane >= 2, ea, 0.0)
                    accB[1] = accB[1] + jnp.where(lane < 2, eb, 0.0)
                    accB[2] = accB[2] + jnp.where(lane >= 2, eb, 0.0)
                    elA = elA + jnp.where(lane == 1, ea, 0.0)
                    elB = elB + jnp.where(lane == 1, eb, 0.0)
                elif k < 62:
                    obuf[pl.ds(obaseA + 16 * k - 2, 16)] = za
                    obuf[pl.ds(obaseB + 16 * k - 2, 16)] = zb
                    accA[2] = accA[2] + ea
                    accB[2] = accB[2] + eb
                else:
                    plsc.store_scatter(obuf, [obaseA + 985 + lane], za,
                                       mask=(lane >= 5) & (lane <= 14))
                    plsc.store_scatter(obuf, [obaseB + 985 + lane], zb,
                                       mask=(lane >= 5) & (lane <= 14))
                    accA[2] = accA[2] + jnp.where(lane >= 5, ea, 0.0)
                    accB[2] = accB[2] + jnp.where(lane >= 5, eb, 0.0)
                    elA = elA + jnp.where(lane == 15, ea, 0.0)
                    elB = elB + jnp.where(lane == 15, eb, 0.0)
                cur = nxt

            sA1, sA2, sA3 = (jnp.sum(a) for a in accA)
            sB1, sB2, sB3 = (jnp.sum(a) for a in accB)
            svA = jnp.where(lane == 7, sA1, jnp.where(lane == 1, sA2, sA3))
            svB = jnp.where(lane == 7, sB1, jnp.where(lane == 1, sB2, sB3))
            rnA = 3.0 - jnp.sum(elA / svA)
            rnB = 3.0 - jnp.sum(elB / svB)
            packed = jnp.where(
                lane == 0, sA1 * rnA,
                jnp.where(lane == 1, sA2 * rnA,
                          jnp.where(lane == 2, sA3 * rnA,
                                    jnp.where(lane == 3, sB1 * rnB,
                                              jnp.where(lane == 4, sB2 * rnB,
                                                        sB3 * rnB)))))
            cvec = _vlog(packed)
            cA1 = jnp.sum(jnp.where(lane == 0, cvec, 0.0))
            cA2 = jnp.sum(jnp.where(lane == 1, cvec, 0.0))
            cA3 = jnp.sum(jnp.where(lane == 2, cvec, 0.0))
            cB1 = jnp.sum(jnp.where(lane == 3, cvec, 0.0))
            cB2 = jnp.sum(jnp.where(lane == 4, cvec, 0.0))
            cB3 = jnp.sum(jnp.where(lane == 5, cvec, 0.0))
            mixA = jnp.where(lane < 7, cA1, cA2)
            mixB = jnp.where(lane < 7, cB1, cB2)

            def bcorr(k, cA, cB):
                pA = obaseA + 16 * k
                pB = obaseB + 16 * k
                va = obuf[pl.ds(pA, 16)]
                vb = obuf[pl.ds(pB, 16)]
                obuf[pl.ds(pA, 16)] = va - cA
                obuf[pl.ds(pB, 16)] = vb - cB

            for k in range(62):
                if k < 24:
                    bcorr(k, cA1, cB1)
                elif k == 24:
                    bcorr(k, mixA, mixB)
                elif k < 54:
                    bcorr(k, cA2, cB2)
                else:
                    bcorr(k, cA3, cB3)
            pA = obaseA + 984
            pB = obaseB + 984
            va = obuf[pl.ds(pA, 16)]
            vb = obuf[pl.ds(pB, 16)]
            plsc.store_scatter(obuf, [pA + lane], va - cA3, mask=lane >= 8)
            plsc.store_scatter(obuf, [pB + lane], vb - cB3, mask=lane >= 8)

    pltpu.async_copy(in_src(0), xb0, isem0)
    pltpu.async_copy(in_src(1), xb1, isem1)

    def outer(gg, carry):
        for par in range(2):
            g = gg * 2 + par
            xbuf, obuf = xbufs[par], obufs[par]
            isem, osem = isems[par], osems[par]
            pltpu.make_async_copy(in_src(g), xbuf, isem).wait()

            @pl.when(gg >= 1)
            def _():
                pltpu.make_async_copy(obuf, out_dst(g), osem).wait()

            compute_chunk(xbuf, obuf)
            pltpu.async_copy(obuf, out_dst(g), osem)

            @pl.when(gg < (_NCH // 2 - 1))
            def _():
                pltpu.async_copy(in_src(g + 2), xbuf, isem)
        return carry

    lax.fori_loop(0, _NCH // 2, outer, 0)
    pltpu.make_async_copy(ob0, out_dst(_NCH - 2), osem0).wait()
    pltpu.make_async_copy(ob1, out_dst(_NCH - 1), osem1).wait()


@jax.jit
def _run_sc(x, t, b):
    mesh = plsc.VectorSubcoreMesh(core_axis_name="c", subcore_axis_name="s")
    f = functools.partial(
        pl.kernel,
        mesh=mesh,
        compiler_params=pltpu.CompilerParams(needs_layout_passes=False),
        out_type=jax.ShapeDtypeStruct((_B * _NOUT,), jnp.float32),
        scratch_types=[
            pltpu.VMEM((_CHUNK * _N,), jnp.float32),
            pltpu.VMEM((_CHUNK * _N,), jnp.float32),
            pltpu.VMEM((_CHUNK * _NOUT,), jnp.float32),
            pltpu.VMEM((_CHUNK * _NOUT,), jnp.float32),
            pltpu.VMEM((_NPAD,), jnp.float32),
            pltpu.VMEM((_NPAD,), jnp.float32),
            pltpu.SemaphoreType.DMA,
            pltpu.SemaphoreType.DMA,
            pltpu.SemaphoreType.DMA,
            pltpu.SemaphoreType.DMA,
        ],
    )(_sc_kernel_body)
    return f(x.reshape(-1), t, b).reshape(_B, _NOUT)


def kernel(x, manyshotTemp, mediumshotTemp, fewshotTemp, manyshotBias,
           mediumshotBias, fewshotBias, many_mask, med_mask, few_mask):
    pad = jnp.zeros((1, _NPAD - _N), jnp.float32)
    t = jnp.concatenate([manyshotTemp, mediumshotTemp, fewshotTemp, pad], axis=1)[0]
    b = jnp.concatenate([manyshotBias, mediumshotBias, fewshotBias, pad], axis=1)[0]
    return _run_sc(x, t, b)


# speedup vs baseline: 2.4320x; 1.5823x over previous
import functools
import jax
import jax.numpy as jnp
from jax import lax
from jax.experimental import pallas as pl
from jax.experimental.pallas import tpu as pltpu
from jax.experimental.pallas import tpu_sc as plsc

_B = 16384
_N = 1003
_NOUT = 1000
_NPAD = 1008
_NW = 32
_RPW = _B // _NW
_CHUNK = 16
_NCH = _RPW // _CHUNK

_LN2 = 0.6931471805599453
_SQRT2 = 1.4142135623730951


def _vlog(v):
    bits = lax.bitcast_convert_type(v, jnp.int32)
    e = (bits >> 23) - 127
    m = lax.bitcast_convert_type((bits & 0x007FFFFF) | 0x3F800000, jnp.float32)
    big = m > _SQRT2
    m = jnp.where(big, m * 0.5, m)
    e = e + jnp.where(big, 1, 0)
    t = (m - 1.0) / (m + 1.0)
    t2 = t * t
    p = t * (2.0 + t2 * (2.0 / 3.0 + t2 * (2.0 / 5.0 + t2 * (2.0 / 7.0 + t2 * (2.0 / 9.0)))))
    return e.astype(jnp.float32) * _LN2 + p


_A_OFFS = [16 * k for k in range(62)] + [987]


def _sc_kernel_body(x_hbm, t_hbm, b_hbm, out_hbm,
                    xb0, xb1, ob0, ob1, tbuf, bbuf,
                    isem0, isem1, osem0, osem1):
    wid = lax.axis_index("s") * 2 + lax.axis_index("c")
    pltpu.sync_copy(t_hbm, tbuf)
    pltpu.sync_copy(b_hbm, bbuf)
    lane = lax.iota(jnp.int32, 16)
    xbufs = (xb0, xb1)
    obufs = (ob0, ob1)
    isems = (isem0, isem1)
    osems = (osem0, osem1)

    def in_src(g):
        row0 = wid * _RPW + g * _CHUNK
        return x_hbm.at[pl.ds(row0, _CHUNK)]

    def out_dst(g):
        row0 = wid * _RPW + g * _CHUNK
        return out_hbm.at[pl.ds(row0, _CHUNK)]

    def compute_chunk(xbuf, obuf):
        @plsc.parallel_loop(0, _CHUNK // 2, 1, unroll=1)
        def pair_body(q):
            rA = q
            rB = q + 8
            rAv = jnp.full((16,), 0, jnp.int32) + rA
            rBv = jnp.full((16,), 0, jnp.int32) + rB

            def lds(k):
                off = _A_OFFS[k]
                return (tbuf[pl.ds(off, 16)], bbuf[pl.ds(off, 16)],
                        xbuf[rA, pl.ds(off, 16)],
                        xbuf[rB, pl.ds(off, 16)])

            def sc_store(rv, col0, val, mask=None):
                plsc.store_scatter(obuf, [rv, col0 + lane], val, mask=mask)

            accA = [jnp.zeros((16,), jnp.float32) for _ in range(3)]
            accB = [jnp.zeros((16,), jnp.float32) for _ in range(3)]
            elA = jnp.zeros((16,), jnp.float32)
            elB = jnp.zeros((16,), jnp.float32)

            cur = lds(0)
            for k in range(63):
                nxt = lds(k + 1) if k < 62 else cur
                tv, bv, xa, xc = cur
                za = xa * tv + bv
                zb = xc * tv + bv
                ea = jnp.exp(za)
                eb = jnp.exp(zb)
                if k < 24:
                    obuf[rA, pl.ds(16 * k, 16)] = za
                    obuf[rB, pl.ds(16 * k, 16)] = zb
                    accA[0] = accA[0] + ea
                    accB[0] = accB[0] + eb
                elif k == 24:
                    sc_store(rAv, 384, za, lane < 7)
                    sc_store(rAv, 383, za, lane >= 8)
                    sc_store(rBv, 384, zb, lane < 7)
                    sc_store(rBv, 383, zb, lane >= 8)
                    accA[0] = accA[0] + jnp.where(lane < 8, ea, 0.0)
                    accA[1] = accA[1] + jnp.where(lane >= 8, ea, 0.0)
                    accB[0] = accB[0] + jnp.where(lane < 8, eb, 0.0)
                    accB[1] = accB[1] + jnp.where(lane >= 8, eb, 0.0)
                    elA = elA + jnp.where(lane == 7, ea, 0.0)
                    elB = elB + jnp.where(lane == 7, eb, 0.0)
                elif k < 54:
                    sc_store(rAv, 16 * k - 1, za)
                    sc_store(rBv, 16 * k - 1, zb)
                    accA[1] = accA[1] + ea
                    accB[1] = accB[1] + eb
                elif k == 54:
                    colx = jnp.where(lane == 0, 863, 862 + lane)
                    plsc.store_scatter(obuf, [rAv, colx], za, mask=lane != 1)
                    plsc.store_scatter(obuf, [rBv, colx], zb, mask=lane != 1)
                    accA[1] = accA[1] + jnp.where(lane < 2, ea, 0.0)
                    accA[2] = accA[2] + jnp.where(lane >= 2, ea, 0.0)
                    accB[1] = accB[1] + jnp.where(lane < 2, eb, 0.0)
                    accB[2] = accB[2] + jnp.where(lane >= 2, eb, 0.0)
                    elA = elA + jnp.where(lane == 1, ea, 0.0)
                    elB = elB + jnp.where(lane == 1, eb, 0.0)
                elif k < 62:
                    sc_store(rAv, 16 * k - 2, za)
                    sc_store(rBv, 16 * k - 2, zb)
                    accA[2] = accA[2] + ea
                    accB[2] = accB[2] + eb
                else:
                    sc_store(rAv, 985, za, (lane >= 5) & (lane <= 14))
                    sc_store(rBv, 985, zb, (lane >= 5) & (lane <= 14))
                    accA[2] = accA[2] + jnp.where(lane >= 5, ea, 0.0)
                    accB[2] = accB[2] + jnp.where(lane >= 5, eb, 0.0)
                    elA = elA + jnp.where(lane == 15, ea, 0.0)
                    elB = elB + jnp.where(lane == 15, eb, 0.0)
                cur = nxt

            sA1, sA2, sA3 = (jnp.sum(a) for a in accA)
            sB1, sB2, sB3 = (jnp.sum(a) for a in accB)
            svA = jnp.where(lane == 7, sA1, jnp.where(lane == 1, sA2, sA3))
            svB = jnp.where(lane == 7, sB1, jnp.where(lane == 1, sB2, sB3))
            rnA = 3.0 - jnp.sum(elA / svA)
            rnB = 3.0 - jnp.sum(elB / svB)
            packed = jnp.where(
                lane == 0, sA1 * rnA,
                jnp.where(lane == 1, sA2 * rnA,
                          jnp.where(lane == 2, sA3 * rnA,
                                    jnp.where(lane == 3, sB1 * rnB,
                                              jnp.where(lane == 4, sB2 * rnB,
                                                        sB3 * rnB)))))
            cvec = _vlog(packed)
            cA1 = jnp.sum(jnp.where(lane == 0, cvec, 0.0))
            cA2 = jnp.sum(jnp.where(lane == 1, cvec, 0.0))
            cA3 = jnp.sum(jnp.where(lane == 2, cvec, 0.0))
            cB1 = jnp.sum(jnp.where(lane == 3, cvec, 0.0))
            cB2 = jnp.sum(jnp.where(lane == 4, cvec, 0.0))
            cB3 = jnp.sum(jnp.where(lane == 5, cvec, 0.0))
            mixA = jnp.where(lane < 7, cA1, cA2)
            mixB = jnp.where(lane < 7, cB1, cB2)

            def bcorr(k, cA, cB):
                p = 16 * k
                va = obuf[rA, pl.ds(p, 16)]
                vb = obuf[rB, pl.ds(p, 16)]
                obuf[rA, pl.ds(p, 16)] = va - cA
                obuf[rB, pl.ds(p, 16)] = vb - cB

            for k in range(62):
                if k < 24:
                    bcorr(k, cA1, cB1)
                elif k == 24:
                    bcorr(k, mixA, mixB)
                elif k < 54:
                    bcorr(k, cA2, cB2)
                else:
                    bcorr(k, cA3, cB3)
            va = obuf[rA, pl.ds(984, 16)]
            vb = obuf[rB, pl.ds(984, 16)]
            sc_store(rAv, 984, va - cA3, lane >= 8)
            sc_store(rBv, 984, vb - cB3, lane >= 8)

    pltpu.async_copy(in_src(0), xb0, isem0)
    pltpu.async_copy(in_src(1), xb1, isem1)

    def outer(gg, carry):
        for par in range(2):
            g = gg * 2 + par
            xbuf, obuf = xbufs[par], obufs[par]
            isem, osem = isems[par], osems[par]
            pltpu.make_async_copy(in_src(g), xbuf, isem).wait()

            @pl.when(gg >= 1)
            def _():
                pltpu.make_async_copy(obuf, out_dst(g), osem).wait()

            compute_chunk(xbuf, obuf)
            pltpu.async_copy(obuf, out_dst(g), osem)

            @pl.when(gg < (_NCH // 2 - 1))
            def _():
                pltpu.async_copy(in_src(g + 2), xbuf, isem)
        return carry

    lax.fori_loop(0, _NCH // 2, outer, 0)
    pltpu.make_async_copy(ob0, out_dst(_NCH - 2), osem0).wait()
    pltpu.make_async_copy(ob1, out_dst(_NCH - 1), osem1).wait()


@jax.jit
def _run_sc(x, t, b):
    mesh = plsc.VectorSubcoreMesh(core_axis_name="c", subcore_axis_name="s")
    f = functools.partial(
        pl.kernel,
        mesh=mesh,
        compiler_params=pltpu.CompilerParams(needs_layout_passes=False),
        out_type=jax.ShapeDtypeStruct((_B, _NOUT), jnp.float32),
        scratch_types=[
            pltpu.VMEM((_CHUNK, _N), jnp.float32),
            pltpu.VMEM((_CHUNK, _N), jnp.float32),
            pltpu.VMEM((_CHUNK, _NOUT), jnp.float32),
            pltpu.VMEM((_CHUNK, _NOUT), jnp.float32),
            pltpu.VMEM((_NPAD,), jnp.float32),
            pltpu.VMEM((_NPAD,), jnp.float32),
            pltpu.SemaphoreType.DMA,
            pltpu.SemaphoreType.DMA,
            pltpu.SemaphoreType.DMA,
            pltpu.SemaphoreType.DMA,
        ],
    )(_sc_kernel_body)
    return f(x, t, b)


def kernel(x, manyshotTemp, mediumshotTemp, fewshotTemp, manyshotBias,
           mediumshotBias, fewshotBias, many_mask, med_mask, few_mask):
    pad = jnp.zeros((1, _NPAD - _N), jnp.float32)
    t = jnp.concatenate([manyshotTemp, mediumshotTemp, fewshotTemp, pad], axis=1)[0]
    b = jnp.concatenate([manyshotBias, mediumshotBias, fewshotBias, pad], axis=1)[0]
    return _run_sc(x, t, b)


# speedup vs baseline: 2.9387x; 1.2083x over previous
import functools
import jax
import jax.numpy as jnp
from jax import lax
from jax.experimental import pallas as pl
from jax.experimental.pallas import tpu as pltpu
from jax.experimental.pallas import tpu_sc as plsc

_B = 16384
_N = 1003
_NOUT = 1000
_NPAD = 1008
_NW = 32
_SC_ROWS = 6144
_TC_ROWS = _B - _SC_ROWS
_TC_BLK = 512
_RPW = _SC_ROWS // _NW
_CHUNK = 16
_NCH = _RPW // _CHUNK

_LN2 = 0.6931471805599453
_SQRT2 = 1.4142135623730951


def _vlog(v):
    bits = lax.bitcast_convert_type(v, jnp.int32)
    e = (bits >> 23) - 127
    m = lax.bitcast_convert_type((bits & 0x007FFFFF) | 0x3F800000, jnp.float32)
    big = m > _SQRT2
    m = jnp.where(big, m * 0.5, m)
    e = e + jnp.where(big, 1, 0)
    t = (m - 1.0) / (m + 1.0)
    t2 = t * t
    p = t * (2.0 + t2 * (2.0 / 3.0 + t2 * (2.0 / 5.0 + t2 * (2.0 / 7.0 + t2 * (2.0 / 9.0)))))
    return e.astype(jnp.float32) * _LN2 + p


_A_OFFS = [16 * k for k in range(62)] + [987]


def _sc_kernel_body(x_hbm, t_hbm, b_hbm, out_hbm,
                    xb0, xb1, ob0, ob1, tbuf, bbuf,
                    isem0, isem1, osem0, osem1):
    wid = lax.axis_index("s") * 2 + lax.axis_index("c")
    pltpu.sync_copy(t_hbm, tbuf)
    pltpu.sync_copy(b_hbm, bbuf)
    lane = lax.iota(jnp.int32, 16)
    xbufs = (xb0, xb1)
    obufs = (ob0, ob1)
    isems = (isem0, isem1)
    osems = (osem0, osem1)

    def in_src(g):
        row0 = wid * _RPW + g * _CHUNK
        return x_hbm.at[pl.ds(row0, _CHUNK)]

    def out_dst(g):
        row0 = wid * _RPW + g * _CHUNK
        return out_hbm.at[pl.ds(row0, _CHUNK)]

    def compute_chunk(xbuf, obuf):
        @plsc.parallel_loop(0, _CHUNK // 2, 1, unroll=1)
        def pair_body(q):
            rA = q
            rB = q + 8
            rAv = jnp.full((16,), 0, jnp.int32) + rA
            rBv = jnp.full((16,), 0, jnp.int32) + rB

            def lds(k):
                off = _A_OFFS[k]
                return (tbuf[pl.ds(off, 16)], bbuf[pl.ds(off, 16)],
                        xbuf[rA, pl.ds(off, 16)],
                        xbuf[rB, pl.ds(off, 16)])

            def sc_store(rv, col0, val, mask=None):
                plsc.store_scatter(obuf, [rv, col0 + lane], val, mask=mask)

            accA = [jnp.zeros((16,), jnp.float32) for _ in range(3)]
            accB = [jnp.zeros((16,), jnp.float32) for _ in range(3)]
            elA = jnp.zeros((16,), jnp.float32)
            elB = jnp.zeros((16,), jnp.float32)

            cur = lds(0)
            for k in range(63):
                nxt = lds(k + 1) if k < 62 else cur
                tv, bv, xa, xc = cur
                za = xa * tv + bv
                zb = xc * tv + bv
                ea = jnp.exp(za)
                eb = jnp.exp(zb)
                if k < 24:
                    obuf[rA, pl.ds(16 * k, 16)] = za
                    obuf[rB, pl.ds(16 * k, 16)] = zb
                    accA[0] = accA[0] + ea
                    accB[0] = accB[0] + eb
                elif k == 24:
                    sc_store(rAv, 384, za, lane < 7)
                    sc_store(rAv, 383, za, lane >= 8)
                    sc_store(rBv, 384, zb, lane < 7)
                    sc_store(rBv, 383, zb, lane >= 8)
                    accA[0] = accA[0] + jnp.where(lane < 8, ea, 0.0)
                    accA[1] = accA[1] + jnp.where(lane >= 8, ea, 0.0)
                    accB[0] = accB[0] + jnp.where(lane < 8, eb, 0.0)
                    accB[1] = accB[1] + jnp.where(lane >= 8, eb, 0.0)
                    elA = elA + jnp.where(lane == 7, ea, 0.0)
                    elB = elB + jnp.where(lane == 7, eb, 0.0)
                elif k < 54:
                    sc_store(rAv, 16 * k - 1, za)
                    sc_store(rBv, 16 * k - 1, zb)
                    accA[1] = accA[1] + ea
                    accB[1] = accB[1] + eb
                elif k == 54:
                    colx = jnp.where(lane == 0, 863, 862 + lane)
                    plsc.store_scatter(obuf, [rAv, colx], za, mask=lane != 1)
                    plsc.store_scatter(obuf, [rBv, colx], zb, mask=lane != 1)
                    accA[1] = accA[1] + jnp.where(lane < 2, ea, 0.0)
                    accA[2] = accA[2] + jnp.where(lane >= 2, ea, 0.0)
                    accB[1] = accB[1] + jnp.where(lane < 2, eb, 0.0)
                    accB[2] = accB[2] + jnp.where(lane >= 2, eb, 0.0)
                    elA = elA + jnp.where(lane == 1, ea, 0.0)
                    elB = elB + jnp.where(lane == 1, eb, 0.0)
                elif k < 62:
                    sc_store(rAv, 16 * k - 2, za)
                    sc_store(rBv, 16 * k - 2, zb)
                    accA[2] = accA[2] + ea
                    accB[2] = accB[2] + eb
                else:
                    sc_store(rAv, 985, za, (lane >= 5) & (lane <= 14))
                    sc_store(rBv, 985, zb, (lane >= 5) & (lane <= 14))
                    accA[2] = accA[2] + jnp.where(lane >= 5, ea, 0.0)
                    accB[2] = accB[2] + jnp.where(lane >= 5, eb, 0.0)
                    elA = elA + jnp.where(lane == 15, ea, 0.0)
                    elB = elB + jnp.where(lane == 15, eb, 0.0)
                cur = nxt

            sA1, sA2, sA3 = (jnp.sum(a) for a in accA)
            sB1, sB2, sB3 = (jnp.sum(a) for a in accB)
            svA = jnp.where(lane == 7, sA1, jnp.where(lane == 1, sA2, sA3))
            svB = jnp.where(lane == 7, sB1, jnp.where(lane == 1, sB2, sB3))
            rnA = 3.0 - jnp.sum(elA / svA)
            rnB = 3.0 - jnp.sum(elB / svB)
            packed = jnp.where(
                lane == 0, sA1 * rnA,
                jnp.where(lane == 1, sA2 * rnA,
                          jnp.where(lane == 2, sA3 * rnA,
                                    jnp.where(lane == 3, sB1 * rnB,
                                              jnp.where(lane == 4, sB2 * rnB,
                                                        sB3 * rnB)))))
            cvec = _vlog(packed)
            cA1 = jnp.sum(jnp.where(lane == 0, cvec, 0.0))
            cA2 = jnp.sum(jnp.where(lane == 1, cvec, 0.0))
            cA3 = jnp.sum(jnp.where(lane == 2, cvec, 0.0))
            cB1 = jnp.sum(jnp.where(lane == 3, cvec, 0.0))
            cB2 = jnp.sum(jnp.where(lane == 4, cvec, 0.0))
            cB3 = jnp.sum(jnp.where(lane == 5, cvec, 0.0))
            mixA = jnp.where(lane < 7, cA1, cA2)
            mixB = jnp.where(lane < 7, cB1, cB2)

            def bcorr(k, cA, cB):
                p = 16 * k
                va = obuf[rA, pl.ds(p, 16)]
                vb = obuf[rB, pl.ds(p, 16)]
                obuf[rA, pl.ds(p, 16)] = va - cA
                obuf[rB, pl.ds(p, 16)] = vb - cB

            for k in range(62):
                if k < 24:
                    bcorr(k, cA1, cB1)
                elif k == 24:
                    bcorr(k, mixA, mixB)
                elif k < 54:
                    bcorr(k, cA2, cB2)
                else:
                    bcorr(k, cA3, cB3)
            va = obuf[rA, pl.ds(984, 16)]
            vb = obuf[rB, pl.ds(984, 16)]
            sc_store(rAv, 984, va - cA3, lane >= 8)
            sc_store(rBv, 984, vb - cB3, lane >= 8)

    pltpu.async_copy(in_src(0), xb0, isem0)
    pltpu.async_copy(in_src(1), xb1, isem1)

    def outer(gg, carry):
        for par in range(2):
            g = gg * 2 + par
            xbuf, obuf = xbufs[par], obufs[par]
            isem, osem = isems[par], osems[par]
            pltpu.make_async_copy(in_src(g), xbuf, isem).wait()

            @pl.when(gg >= 1)
            def _():
                pltpu.make_async_copy(obuf, out_dst(g), osem).wait()

            compute_chunk(xbuf, obuf)
            pltpu.async_copy(obuf, out_dst(g), osem)

            @pl.when(gg < (_NCH // 2 - 1))
            def _():
                pltpu.async_copy(in_src(g + 2), xbuf, isem)
        return carry

    lax.fori_loop(0, _NCH // 2, outer, 0)
    pltpu.make_async_copy(ob0, out_dst(_NCH - 2), osem0).wait()
    pltpu.make_async_copy(ob1, out_dst(_NCH - 1), osem1).wait()


def _run_sc(x, t, b):
    mesh = plsc.VectorSubcoreMesh(core_axis_name="c", subcore_axis_name="s")
    f = functools.partial(
        pl.kernel,
        mesh=mesh,
        compiler_params=pltpu.CompilerParams(needs_layout_passes=False),
        out_type=jax.ShapeDtypeStruct((_SC_ROWS, _NOUT), jnp.float32),
        scratch_types=[
            pltpu.VMEM((_CHUNK, _N), jnp.float32),
            pltpu.VMEM((_CHUNK, _N), jnp.float32),
            pltpu.VMEM((_CHUNK, _NOUT), jnp.float32),
            pltpu.VMEM((_CHUNK, _NOUT), jnp.float32),
            pltpu.VMEM((_NPAD,), jnp.float32),
            pltpu.VMEM((_NPAD,), jnp.float32),
            pltpu.SemaphoreType.DMA,
            pltpu.SemaphoreType.DMA,
            pltpu.SemaphoreType.DMA,
            pltpu.SemaphoreType.DMA,
        ],
    )(_sc_kernel_body)
    return f(x, t, b)


def _tc_body(x_ref, t_ref, b_ref, o_ref):
    z = x_ref[...] * t_ref[...] + b_ref[...]
    z1 = z[:, 0:392]
    z2 = z[:, 392:866]
    z3 = z[:, 866:_N]
    m1 = jnp.max(z1, axis=1, keepdims=True)
    m2 = jnp.max(z2, axis=1, keepdims=True)
    m3 = jnp.max(z3, axis=1, keepdims=True)
    e1 = jnp.exp(z1 - m1)
    e2 = jnp.exp(z2 - m2)
    e3 = jnp.exp(z3 - m3)
    s1 = jnp.sum(e1, axis=1, keepdims=True)
    s2 = jnp.sum(e2, axis=1, keepdims=True)
    s3 = jnp.sum(e3, axis=1, keepdims=True)
    renorm = 3.0 - e1[:, -1:] / s1 - e2[:, -1:] / s2 - e3[:, -1:] / s3
    lr = jnp.log(renorm)
    c1 = m1 + jnp.log(s1) + lr
    c2 = m2 + jnp.log(s2) + lr
    c3 = m3 + jnp.log(s3) + lr
    o_ref[:, 0:391] = z1[:, :-1] - c1
    o_ref[:, 391:864] = z2[:, :-1] - c2
    o_ref[:, 864:1000] = z3[:, :-1] - c3


def _run_tc(x, t2, b2):
    nblk = _TC_ROWS // _TC_BLK
    off = _SC_ROWS // _TC_BLK
    return pl.pallas_call(
        _tc_body,
        grid=(nblk,),
        in_specs=[
            pl.BlockSpec((_TC_BLK, _N), lambda i: (i + off, 0)),
            pl.BlockSpec((1, _N), lambda i: (0, 0)),
            pl.BlockSpec((1, _N), lambda i: (0, 0)),
        ],
        out_specs=pl.BlockSpec((_TC_BLK, _NOUT), lambda i: (i, 0)),
        out_shape=jax.ShapeDtypeStruct((_TC_ROWS, _NOUT), jnp.float32),
    )(x, t2, b2)


@jax.jit
def _run_hybrid(x, t, b, t2, b2):
    out_sc = _run_sc(x, t, b)
    out_tc = _run_tc(x, t2, b2)
    return jnp.concatenate([out_sc, out_tc], axis=0)


def kernel(x, manyshotTemp, mediumshotTemp, fewshotTemp, manyshotBias,
           mediumshotBias, fewshotBias, many_mask, med_mask, few_mask):
    t2 = jnp.concatenate([manyshotTemp, mediumshotTemp, fewshotTemp], axis=1)
    b2 = jnp.concatenate([manyshotBias, mediumshotBias, fewshotBias], axis=1)
    pad = jnp.zeros((1, _NPAD - _N), jnp.float32)
    t = jnp.concatenate([t2, pad], axis=1)[0]
    b = jnp.concatenate([b2, pad], axis=1)[0]
    return _run_hybrid(x, t, b, t2, b2)


# speedup vs baseline: 2.9793x; 1.0138x over previous
import functools
import jax
import jax.numpy as jnp
from jax import lax
from jax.experimental import pallas as pl
from jax.experimental.pallas import tpu as pltpu
from jax.experimental.pallas import tpu_sc as plsc

_B = 16384
_N = 1003
_NOUT = 1000
_NPAD = 1008
_NW = 32
_SC_ROWS = 8192
_TC_ROWS = _B - _SC_ROWS
_TC_BLK = 512
_RPW = _SC_ROWS // _NW
_CHUNK = 16
_NCH = _RPW // _CHUNK

_LN2 = 0.6931471805599453
_SQRT2 = 1.4142135623730951


def _vlog(v):
    bits = lax.bitcast_convert_type(v, jnp.int32)
    e = (bits >> 23) - 127
    m = lax.bitcast_convert_type((bits & 0x007FFFFF) | 0x3F800000, jnp.float32)
    big = m > _SQRT2
    m = jnp.where(big, m * 0.5, m)
    e = e + jnp.where(big, 1, 0)
    t = (m - 1.0) / (m + 1.0)
    t2 = t * t
    p = t * (2.0 + t2 * (2.0 / 3.0 + t2 * (2.0 / 5.0 + t2 * (2.0 / 7.0 + t2 * (2.0 / 9.0)))))
    return e.astype(jnp.float32) * _LN2 + p


_A_OFFS = [16 * k for k in range(62)] + [987]


def _sc_kernel_body(x_hbm, t_hbm, b_hbm, out_hbm,
                    xb0, xb1, ob0, ob1, tbuf, bbuf,
                    isem0, isem1, osem0, osem1):
    wid = lax.axis_index("s") * 2 + lax.axis_index("c")
    pltpu.sync_copy(t_hbm, tbuf)
    pltpu.sync_copy(b_hbm, bbuf)
    lane = lax.iota(jnp.int32, 16)
    xbufs = (xb0, xb1)
    obufs = (ob0, ob1)
    isems = (isem0, isem1)
    osems = (osem0, osem1)

    def in_src(g):
        row0 = wid * _RPW + g * _CHUNK
        return x_hbm.at[pl.ds(row0, _CHUNK)]

    def out_dst(g):
        row0 = wid * _RPW + g * _CHUNK
        return out_hbm.at[pl.ds(row0, _CHUNK)]

    def compute_chunk(xbuf, obuf):
        @plsc.parallel_loop(0, _CHUNK // 2, 1, unroll=1)
        def pair_body(q):
            rA = q
            rB = q + 8
            rAv = jnp.full((16,), 0, jnp.int32) + rA
            rBv = jnp.full((16,), 0, jnp.int32) + rB

            def lds(k):
                off = _A_OFFS[k]
                return (tbuf[pl.ds(off, 16)], bbuf[pl.ds(off, 16)],
                        xbuf[rA, pl.ds(off, 16)],
                        xbuf[rB, pl.ds(off, 16)])

            def sc_store(rv, col0, val, mask=None):
                plsc.store_scatter(obuf, [rv, col0 + lane], val, mask=mask)

            accA = [jnp.zeros((16,), jnp.float32) for _ in range(3)]
            accB = [jnp.zeros((16,), jnp.float32) for _ in range(3)]
            elA = jnp.zeros((16,), jnp.float32)
            elB = jnp.zeros((16,), jnp.float32)

            cur = lds(0)
            for k in range(63):
                nxt = lds(k + 1) if k < 62 else cur
                tv, bv, xa, xc = cur
                za = xa * tv + bv
                zb = xc * tv + bv
                ea = jnp.exp(za)
                eb = jnp.exp(zb)
                if k < 24:
                    obuf[rA, pl.ds(16 * k, 16)] = za
                    obuf[rB, pl.ds(16 * k, 16)] = zb
                    accA[0] = accA[0] + ea
                    accB[0] = accB[0] + eb
                elif k == 24:
                    sc_store(rAv, 384, za, lane < 7)
                    sc_store(rAv, 383, za, lane >= 8)
                    sc_store(rBv, 384, zb, lane < 7)
                    sc_store(rBv, 383, zb, lane >= 8)
                    accA[0] = accA[0] + jnp.where(lane < 8, ea, 0.0)
                    accA[1] = accA[1] + jnp.where(lane >= 8, ea, 0.0)
                    accB[0] = accB[0] + jnp.where(lane < 8, eb, 0.0)
                    accB[1] = accB[1] + jnp.where(lane >= 8, eb, 0.0)
                    elA = elA + jnp.where(lane == 7, ea, 0.0)
                    elB = elB + jnp.where(lane == 7, eb, 0.0)
                elif k < 54:
                    sc_store(rAv, 16 * k - 1, za)
                    sc_store(rBv, 16 * k - 1, zb)
                    accA[1] = accA[1] + ea
                    accB[1] = accB[1] + eb
                elif k == 54:
                    colx = jnp.where(lane == 0, 863, 862 + lane)
                    plsc.store_scatter(obuf, [rAv, colx], za, mask=lane != 1)
                    plsc.store_scatter(obuf, [rBv, colx], zb, mask=lane != 1)
                    accA[1] = accA[1] + jnp.where(lane < 2, ea, 0.0)
                    accA[2] = accA[2] + jnp.where(lane >= 2, ea, 0.0)
                    accB[1] = accB[1] + jnp.where(lane < 2, eb, 0.0)
                    accB[2] = accB[2] + jnp.where(lane >= 2, eb, 0.0)
                    elA = elA + jnp.where(lane == 1, ea, 0.0)
                    elB = elB + jnp.where(lane == 1, eb, 0.0)
                elif k < 62:
                    sc_store(rAv, 16 * k - 2, za)
                    sc_store(rBv, 16 * k - 2, zb)
                    accA[2] = accA[2] + ea
                    accB[2] = accB[2] + eb
                else:
                    sc_store(rAv, 985, za, (lane >= 5) & (lane <= 14))
                    sc_store(rBv, 985, zb, (lane >= 5) & (lane <= 14))
                    accA[2] = accA[2] + jnp.where(lane >= 5, ea, 0.0)
                    accB[2] = accB[2] + jnp.where(lane >= 5, eb, 0.0)
                    elA = elA + jnp.where(lane == 15, ea, 0.0)
                    elB = elB + jnp.where(lane == 15, eb, 0.0)
                cur = nxt

            sA1, sA2, sA3 = (jnp.sum(a) for a in accA)
            sB1, sB2, sB3 = (jnp.sum(a) for a in accB)
            svA = jnp.where(lane == 7, sA1, jnp.where(lane == 1, sA2, sA3))
            svB = jnp.where(lane == 7, sB1, jnp.where(lane == 1, sB2, sB3))
            rnA = 3.0 - jnp.sum(elA / svA)
            rnB = 3.0 - jnp.sum(elB / svB)
            packed = jnp.where(
                lane == 0, sA1 * rnA,
                jnp.where(lane == 1, sA2 * rnA,
                          jnp.where(lane == 2, sA3 * rnA,
                                    jnp.where(lane == 3, sB1 * rnB,
                                              jnp.where(lane == 4, sB2 * rnB,
                                                        sB3 * rnB)))))
            cvec = _vlog(packed)
            cA1 = jnp.sum(jnp.where(lane == 0, cvec, 0.0))
            cA2 = jnp.sum(jnp.where(lane == 1, cvec, 0.0))
            cA3 = jnp.sum(jnp.where(lane == 2, cvec, 0.0))
            cB1 = jnp.sum(jnp.where(lane == 3, cvec, 0.0))
            cB2 = jnp.sum(jnp.where(lane == 4, cvec, 0.0))
            cB3 = jnp.sum(jnp.where(lane == 5, cvec, 0.0))
            mixA = jnp.where(lane < 7, cA1, cA2)
            mixB = jnp.where(lane < 7, cB1, cB2)

            def bcorr(k, cA, cB):
                p = 16 * k
                va = obuf[rA, pl.ds(p, 16)]
                vb = obuf[rB, pl.ds(p, 16)]
                obuf[rA, pl.ds(p, 16)] = va - cA
                obuf[rB, pl.ds(p, 16)] = vb - cB

            for k in range(62):
                if k < 24:
                    bcorr(k, cA1, cB1)
                elif k == 24:
                    bcorr(k, mixA, mixB)
                elif k < 54:
                    bcorr(k, cA2, cB2)
                else:
                    bcorr(k, cA3, cB3)
            va = obuf[rA, pl.ds(984, 16)]
            vb = obuf[rB, pl.ds(984, 16)]
            sc_store(rAv, 984, va - cA3, lane >= 8)
            sc_store(rBv, 984, vb - cB3, lane >= 8)

    pltpu.async_copy(in_src(0), xb0, isem0)
    pltpu.async_copy(in_src(1), xb1, isem1)

    def outer(gg, carry):
        for par in range(2):
            g = gg * 2 + par
            xbuf, obuf = xbufs[par], obufs[par]
            isem, osem = isems[par], osems[par]
            pltpu.make_async_copy(in_src(g), xbuf, isem).wait()

            @pl.when(gg >= 1)
            def _():
                pltpu.make_async_copy(obuf, out_dst(g), osem).wait()

            compute_chunk(xbuf, obuf)
            pltpu.async_copy(obuf, out_dst(g), osem)

            @pl.when(gg < (_NCH // 2 - 1))
            def _():
                pltpu.async_copy(in_src(g + 2), xbuf, isem)
        return carry

    lax.fori_loop(0, _NCH // 2, outer, 0)
    pltpu.make_async_copy(ob0, out_dst(_NCH - 2), osem0).wait()
    pltpu.make_async_copy(ob1, out_dst(_NCH - 1), osem1).wait()


def _run_sc(x, t, b):
    mesh = plsc.VectorSubcoreMesh(core_axis_name="c", subcore_axis_name="s")
    f = functools.partial(
        pl.kernel,
        mesh=mesh,
        compiler_params=pltpu.CompilerParams(needs_layout_passes=False),
        out_type=jax.ShapeDtypeStruct((_SC_ROWS, _NOUT), jnp.float32),
        scratch_types=[
            pltpu.VMEM((_CHUNK, _N), jnp.float32),
            pltpu.VMEM((_CHUNK, _N), jnp.float32),
            pltpu.VMEM((_CHUNK, _NOUT), jnp.float32),
            pltpu.VMEM((_CHUNK, _NOUT), jnp.float32),
            pltpu.VMEM((_NPAD,), jnp.float32),
            pltpu.VMEM((_NPAD,), jnp.float32),
            pltpu.SemaphoreType.DMA,
            pltpu.SemaphoreType.DMA,
            pltpu.SemaphoreType.DMA,
            pltpu.SemaphoreType.DMA,
        ],
    )(_sc_kernel_body)
    return f(x, t, b)


def _tc_body(x_ref, t_ref, b_ref, o_ref):
    z = x_ref[...] * t_ref[...] + b_ref[...]
    z1 = z[:, 0:392]
    z2 = z[:, 392:866]
    z3 = z[:, 866:_N]
    m1 = jnp.max(z1, axis=1, keepdims=True)
    m2 = jnp.max(z2, axis=1, keepdims=True)
    m3 = jnp.max(z3, axis=1, keepdims=True)
    e1 = jnp.exp(z1 - m1)
    e2 = jnp.exp(z2 - m2)
    e3 = jnp.exp(z3 - m3)
    s1 = jnp.sum(e1, axis=1, keepdims=True)
    s2 = jnp.sum(e2, axis=1, keepdims=True)
    s3 = jnp.sum(e3, axis=1, keepdims=True)
    renorm = 3.0 - e1[:, -1:] / s1 - e2[:, -1:] / s2 - e3[:, -1:] / s3
    lr = jnp.log(renorm)
    c1 = m1 + jnp.log(s1) + lr
    c2 = m2 + jnp.log(s2) + lr
    c3 = m3 + jnp.log(s3) + lr
    o_ref[:, 0:391] = z1[:, :-1] - c1
    o_ref[:, 391:864] = z2[:, :-1] - c2
    o_ref[:, 864:1000] = z3[:, :-1] - c3


def _run_tc(x, t2, b2):
    nblk = _TC_ROWS // _TC_BLK
    off = _SC_ROWS // _TC_BLK
    return pl.pallas_call(
        _tc_body,
        grid=(nblk,),
        in_specs=[
            pl.BlockSpec((_TC_BLK, _N), lambda i: (i + off, 0)),
            pl.BlockSpec((1, _N), lambda i: (0, 0)),
            pl.BlockSpec((1, _N), lambda i: (0, 0)),
        ],
        out_specs=pl.BlockSpec((_TC_BLK, _NOUT), lambda i: (i + off, 0)),
        out_shape=jax.ShapeDtypeStruct((_B, _NOUT), jnp.float32),
    )(x, t2, b2)


@jax.jit
def _run_hybrid(x, t, b, t2, b2):
    out_sc = _run_sc(x, t, b)
    out_full = _run_tc(x, t2, b2)
    return lax.dynamic_update_slice(out_full, out_sc, (0, 0))


def kernel(x, manyshotTemp, mediumshotTemp, fewshotTemp, manyshotBias,
           mediumshotBias, fewshotBias, many_mask, med_mask, few_mask):
    t2 = jnp.concatenate([manyshotTemp, mediumshotTemp, fewshotTemp], axis=1)
    b2 = jnp.concatenate([manyshotBias, mediumshotBias, fewshotBias], axis=1)
    pad = jnp.zeros((1, _NPAD - _N), jnp.float32)
    t = jnp.concatenate([t2, pad], axis=1)[0]
    b = jnp.concatenate([b2, pad], axis=1)[0]
    return _run_hybrid(x, t, b, t2, b2)


# speedup vs baseline: 3.2025x; 1.0749x over previous
import functools
import jax
import jax.numpy as jnp
from jax import lax
from jax.experimental import pallas as pl
from jax.experimental.pallas import tpu as pltpu
from jax.experimental.pallas import tpu_sc as plsc

_B = 16384
_N = 1003
_NOUT = 1000
_NPAD = 1008
_NW = 32
_SC_ROWS = 8192
_TC_ROWS = _B - _SC_ROWS
_TC_BLK = 512
_RPW = _SC_ROWS // _NW
_CHUNK = 16
_NCH = _RPW // _CHUNK

_LN2 = 0.6931471805599453
_SQRT2 = 1.4142135623730951


def _vlog(v):
    bits = lax.bitcast_convert_type(v, jnp.int32)
    e = (bits >> 23) - 127
    m = lax.bitcast_convert_type((bits & 0x007FFFFF) | 0x3F800000, jnp.float32)
    big = m > _SQRT2
    m = jnp.where(big, m * 0.5, m)
    e = e + jnp.where(big, 1, 0)
    t = (m - 1.0) / (m + 1.0)
    t2 = t * t
    p = t * (2.0 + t2 * (2.0 / 3.0 + t2 * (2.0 / 5.0 + t2 * (2.0 / 7.0 + t2 * (2.0 / 9.0)))))
    return e.astype(jnp.float32) * _LN2 + p


_A_OFFS = [16 * k for k in range(62)] + [987]


def _sc_kernel_body(x_hbm, t_hbm, b_hbm, out_hbm,
                    xb0, xb1, ob0, ob1, tbuf, bbuf,
                    isem0, isem1, osem0, osem1):
    wid = lax.axis_index("s") * 2 + lax.axis_index("c")
    pltpu.sync_copy(t_hbm, tbuf)
    pltpu.sync_copy(b_hbm, bbuf)
    lane = lax.iota(jnp.int32, 16)
    xbufs = (xb0, xb1)
    obufs = (ob0, ob1)
    isems = (isem0, isem1)
    osems = (osem0, osem1)

    def in_src(g):
        row0 = wid * _RPW + g * _CHUNK
        return x_hbm.at[pl.ds(row0, _CHUNK)]

    def out_dst(g):
        row0 = wid * _RPW + g * _CHUNK
        return out_hbm.at[pl.ds(row0, _CHUNK)]

    def compute_chunk(xbuf, cbuf):
        @plsc.parallel_loop(0, _CHUNK // 2, 1, unroll=1)
        def pair_body(q):
            rA = q
            rB = q + 8

            def lds(k):
                off = _A_OFFS[k]
                return (tbuf[pl.ds(off, 16)], bbuf[pl.ds(off, 16)],
                        xbuf[rA, pl.ds(off, 16)],
                        xbuf[rB, pl.ds(off, 16)])

            accA = [jnp.zeros((16,), jnp.float32) for _ in range(3)]
            accB = [jnp.zeros((16,), jnp.float32) for _ in range(3)]
            elA = jnp.zeros((16,), jnp.float32)
            elB = jnp.zeros((16,), jnp.float32)

            cur = lds(0)
            for k in range(63):
                nxt = lds(k + 1) if k < 62 else cur
                tv, bv, xa, xc = cur
                ea = jnp.exp(xa * tv + bv)
                eb = jnp.exp(xc * tv + bv)
                if k < 24:
                    accA[0] = accA[0] + ea
                    accB[0] = accB[0] + eb
                elif k == 24:
                    accA[0] = accA[0] + jnp.where(lane < 8, ea, 0.0)
                    accA[1] = accA[1] + jnp.where(lane >= 8, ea, 0.0)
                    accB[0] = accB[0] + jnp.where(lane < 8, eb, 0.0)
                    accB[1] = accB[1] + jnp.where(lane >= 8, eb, 0.0)
                    elA = elA + jnp.where(lane == 7, ea, 0.0)
                    elB = elB + jnp.where(lane == 7, eb, 0.0)
                elif k < 54:
                    accA[1] = accA[1] + ea
                    accB[1] = accB[1] + eb
                elif k == 54:
                    accA[1] = accA[1] + jnp.where(lane < 2, ea, 0.0)
                    accA[2] = accA[2] + jnp.where(lane >= 2, ea, 0.0)
                    accB[1] = accB[1] + jnp.where(lane < 2, eb, 0.0)
                    accB[2] = accB[2] + jnp.where(lane >= 2, eb, 0.0)
                    elA = elA + jnp.where(lane == 1, ea, 0.0)
                    elB = elB + jnp.where(lane == 1, eb, 0.0)
                elif k < 62:
                    accA[2] = accA[2] + ea
                    accB[2] = accB[2] + eb
                else:
                    accA[2] = accA[2] + jnp.where(lane >= 5, ea, 0.0)
                    accB[2] = accB[2] + jnp.where(lane >= 5, eb, 0.0)
                    elA = elA + jnp.where(lane == 15, ea, 0.0)
                    elB = elB + jnp.where(lane == 15, eb, 0.0)
                cur = nxt

            sA1, sA2, sA3 = (jnp.sum(a) for a in accA)
            sB1, sB2, sB3 = (jnp.sum(a) for a in accB)
            svA = jnp.where(lane == 7, sA1, jnp.where(lane == 1, sA2, sA3))
            svB = jnp.where(lane == 7, sB1, jnp.where(lane == 1, sB2, sB3))
            rnA = 3.0 - jnp.sum(elA / svA)
            rnB = 3.0 - jnp.sum(elB / svB)
            packedA = jnp.where(lane == 0, sA1 * rnA,
                                jnp.where(lane == 1, sA2 * rnA, sA3 * rnA))
            packedB = jnp.where(lane == 0, sB1 * rnB,
                                jnp.where(lane == 1, sB2 * rnB, sB3 * rnB))
            cbuf[rA, :] = _vlog(packedA)
            cbuf[rB, :] = _vlog(packedB)

    pltpu.async_copy(in_src(0), xb0, isem0)
    pltpu.async_copy(in_src(1), xb1, isem1)

    def outer(gg, carry):
        for par in range(2):
            g = gg * 2 + par
            xbuf, obuf = xbufs[par], obufs[par]
            isem, osem = isems[par], osems[par]
            pltpu.make_async_copy(in_src(g), xbuf, isem).wait()

            @pl.when(gg >= 1)
            def _():
                pltpu.make_async_copy(obuf, out_dst(g), osem).wait()

            compute_chunk(xbuf, obuf)
            pltpu.async_copy(obuf, out_dst(g), osem)

            @pl.when(gg < (_NCH // 2 - 1))
            def _():
                pltpu.async_copy(in_src(g + 2), xbuf, isem)
        return carry

    lax.fori_loop(0, _NCH // 2, outer, 0)
    pltpu.make_async_copy(ob0, out_dst(_NCH - 2), osem0).wait()
    pltpu.make_async_copy(ob1, out_dst(_NCH - 1), osem1).wait()


def _run_sc(x, t, b):
    mesh = plsc.VectorSubcoreMesh(core_axis_name="c", subcore_axis_name="s")
    f = functools.partial(
        pl.kernel,
        mesh=mesh,
        compiler_params=pltpu.CompilerParams(needs_layout_passes=False),
        out_type=jax.ShapeDtypeStruct((_SC_ROWS, 16), jnp.float32),
        scratch_types=[
            pltpu.VMEM((_CHUNK, _N), jnp.float32),
            pltpu.VMEM((_CHUNK, _N), jnp.float32),
            pltpu.VMEM((_CHUNK, 16), jnp.float32),
            pltpu.VMEM((_CHUNK, 16), jnp.float32),
            pltpu.VMEM((_NPAD,), jnp.float32),
            pltpu.VMEM((_NPAD,), jnp.float32),
            pltpu.SemaphoreType.DMA,
            pltpu.SemaphoreType.DMA,
            pltpu.SemaphoreType.DMA,
            pltpu.SemaphoreType.DMA,
        ],
    )(_sc_kernel_body)
    return f(x, t, b)


def _tc_body(x_ref, t_ref, b_ref, o_ref):
    z = x_ref[...] * t_ref[...] + b_ref[...]
    z1 = z[:, 0:392]
    z2 = z[:, 392:866]
    z3 = z[:, 866:_N]
    m1 = jnp.max(z1, axis=1, keepdims=True)
    m2 = jnp.max(z2, axis=1, keepdims=True)
    m3 = jnp.max(z3, axis=1, keepdims=True)
    e1 = jnp.exp(z1 - m1)
    e2 = jnp.exp(z2 - m2)
    e3 = jnp.exp(z3 - m3)
    s1 = jnp.sum(e1, axis=1, keepdims=True)
    s2 = jnp.sum(e2, axis=1, keepdims=True)
    s3 = jnp.sum(e3, axis=1, keepdims=True)
    renorm = 3.0 - e1[:, -1:] / s1 - e2[:, -1:] / s2 - e3[:, -1:] / s3
    lr = jnp.log(renorm)
    c1 = m1 + jnp.log(s1) + lr
    c2 = m2 + jnp.log(s2) + lr
    c3 = m3 + jnp.log(s3) + lr
    o_ref[:, 0:391] = z1[:, :-1] - c1
    o_ref[:, 391:864] = z2[:, :-1] - c2
    o_ref[:, 864:1000] = z3[:, :-1] - c3


def _run_tc(x, t2, b2):
    nblk = _TC_ROWS // _TC_BLK
    off = _SC_ROWS // _TC_BLK
    return pl.pallas_call(
        _tc_body,
        grid=(nblk,),
        in_specs=[
            pl.BlockSpec((_TC_BLK, _N), lambda i: (i + off, 0)),
            pl.BlockSpec((1, _N), lambda i: (0, 0)),
            pl.BlockSpec((1, _N), lambda i: (0, 0)),
        ],
        out_specs=pl.BlockSpec((_TC_BLK, _NOUT), lambda i: (i + off, 0)),
        out_shape=jax.ShapeDtypeStruct((_B, _NOUT), jnp.float32),
    )(x, t2, b2)


def _tc2_body(full_ref, x_ref, t_ref, b_ref, c_ref, o_ref):
    z = x_ref[...] * t_ref[...] + b_ref[...]
    c1 = c_ref[:, 0:1]
    c2 = c_ref[:, 1:2]
    c3 = c_ref[:, 2:3]
    o_ref[:, 0:391] = z[:, 0:391] - c1
    o_ref[:, 391:864] = z[:, 392:865] - c2
    o_ref[:, 864:1000] = z[:, 866:1002] - c3


def _run_tc2(out_full, x, t2, b2, corr):
    nblk = _SC_ROWS // _TC_BLK
    return pl.pallas_call(
        _tc2_body,
        grid=(nblk,),
        in_specs=[
            pl.BlockSpec(memory_space=pltpu.MemorySpace.HBM),
            pl.BlockSpec((_TC_BLK, _N), lambda i: (i, 0)),
            pl.BlockSpec((1, _N), lambda i: (0, 0)),
            pl.BlockSpec((1, _N), lambda i: (0, 0)),
            pl.BlockSpec((_TC_BLK, 16), lambda i: (i, 0)),
        ],
        out_specs=pl.BlockSpec((_TC_BLK, _NOUT), lambda i: (i, 0)),
        out_shape=jax.ShapeDtypeStruct((_B, _NOUT), jnp.float32),
        input_output_aliases={0: 0},
    )(out_full, x, t2, b2, corr)


@jax.jit
def _run_hybrid(x, t, b, t2, b2):
    corr = _run_sc(x, t, b)
    out_full = _run_tc(x, t2, b2)
    return _run_tc2(out_full, x, t2, b2, corr)


def kernel(x, manyshotTemp, mediumshotTemp, fewshotTemp, manyshotBias,
           mediumshotBias, fewshotBias, many_mask, med_mask, few_mask):
    t2 = jnp.concatenate([manyshotTemp, mediumshotTemp, fewshotTemp], axis=1)
    b2 = jnp.concatenate([manyshotBias, mediumshotBias, fewshotBias], axis=1)
    pad = jnp.zeros((1, _NPAD - _N), jnp.float32)
    t = jnp.concatenate([t2, pad], axis=1)[0]
    b = jnp.concatenate([b2, pad], axis=1)[0]
    return _run_hybrid(x, t, b, t2, b2)
